# baseline (speedup 1.0000x reference)
import functools

import jax
import jax.numpy as jnp
from jax import lax
from jax.experimental import pallas as pl
from jax.experimental.pallas import tpu as pltpu
from jax.experimental.pallas import tpu_sc as plsc

VOCAB = 100000
D = 64
N_OUT = 128
B = 4096
L = 50

NC, NS = 2, 16
NW = NC * NS
BPW = B // NW
STEPS = BPW
NBUF = 2

_mesh = plsc.VectorSubcoreMesh(core_axis_name="c", subcore_axis_name="s")


@functools.partial(
    pl.kernel,
    mesh=_mesh,
    compiler_params=pltpu.CompilerParams(use_tc_tiling_on_sc=False),
    out_type=jax.ShapeDtypeStruct((B, D), jnp.float32),
    scratch_types=[
        pltpu.VMEM((BPW, L), jnp.int32),
        pltpu.VMEM((NBUF, L, D), jnp.float32),
        pltpu.VMEM((BPW, D), jnp.float32),
        pltpu.SemaphoreType.DMA,
        pltpu.SemaphoreType.DMA,
    ],
)
def _sc_pool(idx_hbm, table_hbm, out_hbm, idx_v, rows_v, pool_v, sem0, sem1):
    wid = lax.axis_index("s") * NC + lax.axis_index("c")
    sems = (sem0, sem1)

    pltpu.sync_copy(idx_hbm.at[pl.ds(wid * BPW, BPW)], idx_v)

    for b in range(NBUF):
        pltpu.make_async_copy(
            table_hbm.at[idx_v.at[b]], rows_v.at[b], sems[b]
        ).start()

    def pool_one_bag(rows, out_row):
        a0 = rows[0, pl.ds(0, 16)]
        a1 = rows[0, pl.ds(16, 16)]
        a2 = rows[0, pl.ds(32, 16)]
        a3 = rows[0, pl.ds(48, 16)]

        def body(l, acc):
            return (
                jnp.maximum(acc[0], rows[l, pl.ds(0, 16)]),
                jnp.maximum(acc[1], rows[l, pl.ds(16, 16)]),
                jnp.maximum(acc[2], rows[l, pl.ds(32, 16)]),
                jnp.maximum(acc[3], rows[l, pl.ds(48, 16)]),
            )

        a0, a1, a2, a3 = lax.fori_loop(
            1, L, body, (a0, a1, a2, a3), unroll=7
        )
        pool_v[out_row, pl.ds(0, 16)] = a0
        pool_v[out_row, pl.ds(16, 16)] = a1
        pool_v[out_row, pl.ds(32, 16)] = a2
        pool_v[out_row, pl.ds(48, 16)] = a3

    def pair_body(k, _):
        for b in range(NBUF):
            s = NBUF * k + b
            rows = rows_v.at[b]
            pltpu.make_async_copy(
                table_hbm.at[idx_v.at[s]], rows, sems[b]
            ).wait()
            pool_one_bag(rows, s)

            @pl.when(s + NBUF < STEPS)
            def _():
                pltpu.make_async_copy(
                    table_hbm.at[idx_v.at[s + NBUF]], rows, sems[b]
                ).start()

        return 0

    lax.fori_loop(0, STEPS // NBUF, pair_body, 0)

    pltpu.sync_copy(pool_v, out_hbm.at[pl.ds(wid * BPW, BPW)])


def _mm_body(p_ref, w_ref, o_ref):
    o_ref[:] = lax.dot_general(
        p_ref[:], w_ref[:],
        (((1,), (1,)), ((), ())),
        preferred_element_type=jnp.float32,
    )


def kernel(x, table, W_out):
    idx = jnp.squeeze(x, axis=1).astype(jnp.int32)
    pooled = _sc_pool(idx, table)
    out = pl.pallas_call(
        _mm_body,
        out_shape=jax.ShapeDtypeStruct((B, N_OUT), jnp.float32),
    )(pooled, W_out)
    return out


# speedup vs baseline: 1.3058x; 1.3058x over previous
import functools

import jax
import jax.numpy as jnp
from jax import lax
from jax.experimental import pallas as pl
from jax.experimental.pallas import tpu as pltpu
from jax.experimental.pallas import tpu_sc as plsc

VOCAB = 100000
D = 64
N_OUT = 128
B = 4096
L = 50

NC, NS = 2, 16
NW = NC * NS
BPW = B // NW
G = 2
STEPS = BPW // G
NBUF = 4

_mesh = plsc.VectorSubcoreMesh(core_axis_name="c", subcore_axis_name="s")


@functools.partial(
    pl.kernel,
    mesh=_mesh,
    compiler_params=pltpu.CompilerParams(use_tc_tiling_on_sc=False),
    out_type=jax.ShapeDtypeStruct((B, D), jnp.float32),
    scratch_types=[
        pltpu.VMEM((STEPS, G * L), jnp.int32),
        pltpu.VMEM((NBUF, G * L, D), jnp.float32),
        pltpu.VMEM((BPW, D), jnp.float32),
        [pltpu.SemaphoreType.DMA] * NBUF,
    ],
)
def _sc_pool(idx_hbm, table_hbm, out_hbm, idx_v, rows_v, pool_v, sems):
    wid = lax.axis_index("s") * NC + lax.axis_index("c")

    pltpu.sync_copy(idx_hbm.at[wid], idx_v)

    def gather(s, b):
        return pltpu.make_async_copy(
            table_hbm.at[idx_v.at[s]], rows_v.at[b], sems[b]
        )

    for b in range(NBUF - 1):
        gather(b, b).start()

    def pool_one_bag(rows, out_row):
        a0 = rows[0, pl.ds(0, 16)]
        a1 = rows[0, pl.ds(16, 16)]
        a2 = rows[0, pl.ds(32, 16)]
        a3 = rows[0, pl.ds(48, 16)]

        def body(l, acc):
            return (
                jnp.maximum(acc[0], rows[l, pl.ds(0, 16)]),
                jnp.maximum(acc[1], rows[l, pl.ds(16, 16)]),
                jnp.maximum(acc[2], rows[l, pl.ds(32, 16)]),
                jnp.maximum(acc[3], rows[l, pl.ds(48, 16)]),
            )

        a0, a1, a2, a3 = lax.fori_loop(
            1, L, body, (a0, a1, a2, a3), unroll=7
        )
        pool_v[out_row, pl.ds(0, 16)] = a0
        pool_v[out_row, pl.ds(16, 16)] = a1
        pool_v[out_row, pl.ds(32, 16)] = a2
        pool_v[out_row, pl.ds(48, 16)] = a3

    def ring_body(k, _):
        for b in range(NBUF):
            s = NBUF * k + b

            @pl.when(s + NBUF - 1 < STEPS)
            def _():
                gather(s + NBUF - 1, (b + NBUF - 1) % NBUF).start()

            gather(s, b).wait()
            for g in range(G):
                pool_one_bag(rows_v.at[b, pl.ds(g * L, L)], s * G + g)

        return 0

    lax.fori_loop(0, STEPS // NBUF, ring_body, 0)

    pltpu.sync_copy(pool_v, out_hbm.at[pl.ds(wid * BPW, BPW)])


def _mm_body(p_ref, w_ref, o_ref):
    o_ref[:] = lax.dot_general(
        p_ref[:], w_ref[:],
        (((1,), (1,)), ((), ())),
        preferred_element_type=jnp.float32,
    )


def kernel(x, table, W_out):
    idx = jnp.reshape(x.astype(jnp.int32), (NW, STEPS, G * L))
    pooled = _sc_pool(idx, table)
    out = pl.pallas_call(
        _mm_body,
        out_shape=jax.ShapeDtypeStruct((B, N_OUT), jnp.float32),
    )(pooled, W_out)
    return out
